# Initial kernel scaffold; baseline (speedup 1.0000x reference)
import jax
import jax.numpy as jnp
from jax.experimental import pallas as pl


def kernel(x, edge_index, edge_weight, W0, W1, W2):
    raise NotImplementedError("write your pallas kernel here")


# speedup vs baseline: 4.3358x; 4.3358x over previous
import functools

import jax
import jax.numpy as jnp
from jax import lax
from jax.experimental import pallas as pl
from jax.experimental.pallas import tpu as pltpu
from jax.experimental.pallas import tpu_sc as plsc

N_NODES = 10000
D_FEAT = 128
NUM_CLASSES = 64
N_EDGES = 320000

NC = 2
NS = 16
NW = NC * NS
CHUNK = 128
N_CHUNKS = -(-N_EDGES // (NW * CHUNK))
E_PAD = NW * N_CHUNKS * CHUNK
N_PAD = 10240
ROWS_PER_TILE = N_PAD // NS
WB = 128


@functools.lru_cache(maxsize=None)
def _make_spmm(dp: int):
    mesh = plsc.VectorSubcoreMesh(core_axis_name="c", subcore_axis_name="s")

    @functools.partial(
        pl.kernel,
        mesh=mesh,
        out_type=jax.ShapeDtypeStruct((NC, N_PAD, dp), jnp.float32),
        scratch_types=[
            pltpu.VMEM((N_CHUNKS, CHUNK), jnp.int32),
            pltpu.VMEM((N_CHUNKS, CHUNK), jnp.int32),
            pltpu.VMEM((N_CHUNKS, CHUNK), jnp.float32),
            pltpu.VMEM((CHUNK, dp), jnp.float32),
            pltpu.VMEM_SHARED((N_PAD, dp), jnp.float32),
            pltpu.SemaphoreType.DMA,
        ],
    )
    def spmm(x_hbm, src_hbm, dst_hbm, w_hbm, out_hbm,
             src_v, dst_v, w_v, rows_v, acc_s, sem):
        cid = lax.axis_index("c")
        sid = lax.axis_index("s")
        wid = sid * NC + cid

        zero = jnp.zeros((16,), jnp.float32)

        def zrow(i, carry):
            for f in range(dp // 16):
                rows_v[i, pl.ds(16 * f, 16)] = zero
            return carry

        lax.fori_loop(0, CHUNK, zrow, 0)
        for r in range(ROWS_PER_TILE // WB):
            base = sid * ROWS_PER_TILE + r * WB
            pltpu.sync_copy(rows_v.at[pl.ds(0, WB)], acc_s.at[pl.ds(base, WB)])
        plsc.subcore_barrier()

        pltpu.sync_copy(src_hbm.at[wid], src_v)
        pltpu.sync_copy(dst_hbm.at[wid], dst_v)
        pltpu.sync_copy(w_hbm.at[wid], w_v)

        def chunk_body(ci, carry):
            pltpu.async_copy(x_hbm.at[src_v.at[ci]], rows_v, sem).wait()

            def group_body(g, c2):
                wv = w_v[ci, pl.ds(g * 16, 16)]
                for j in range(16):
                    ws = wv[j]
                    e = g * 16 + j
                    for f in range(dp // 16):
                        sl = pl.ds(16 * f, 16)
                        rows_v[e, sl] = rows_v[e, sl] * ws
                return c2

            lax.fori_loop(0, CHUNK // 16, group_body, 0)

            pltpu.sync_copy(rows_v, acc_s.at[dst_v.at[ci]], add=True)
            return carry

        lax.fori_loop(0, N_CHUNKS, chunk_body, 0)
        plsc.subcore_barrier()

        for r in range(ROWS_PER_TILE // WB):
            base = sid * ROWS_PER_TILE + r * WB
            pltpu.sync_copy(acc_s.at[pl.ds(base, WB)], rows_v.at[pl.ds(0, WB)])
            pltpu.sync_copy(rows_v.at[pl.ds(0, WB)], out_hbm.at[cid, pl.ds(base, WB)])

    return spmm


def _spmm(xw, src3, dst3, w3):
    return _make_spmm(xw.shape[1])(xw, src3, dst3, w3)


_BR = 1000


def _mm_body(x_ref, w_ref, o_ref):
    o_ref[...] = lax.dot_general(
        x_ref[...], w_ref[...], (((1,), (1,)), ((), ())),
        preferred_element_type=jnp.float32)


def _matmul(x, w):
    n, d = x.shape
    do = w.shape[0]
    return pl.pallas_call(
        _mm_body,
        grid=(n // _BR,),
        in_specs=[pl.BlockSpec((_BR, d), lambda i: (i, 0)),
                  pl.BlockSpec((do, d), lambda i: (0, 0))],
        out_specs=pl.BlockSpec((_BR, do), lambda i: (i, 0)),
        out_shape=jax.ShapeDtypeStruct((n, do), jnp.float32),
    )(x, w)


def _fuse_body(p_ref, w_ref, o_ref):
    h = jnp.tanh(p_ref[0] + p_ref[1])
    o_ref[...] = lax.dot_general(
        h, w_ref[...], (((1,), (1,)), ((), ())),
        preferred_element_type=jnp.float32)


def _addtanh_matmul(p, w):
    _, n, d = p.shape
    do = w.shape[0]
    return pl.pallas_call(
        _fuse_body,
        grid=(n // _BR,),
        in_specs=[pl.BlockSpec((2, _BR, d), lambda i: (0, i, 0)),
                  pl.BlockSpec((do, d), lambda i: (0, 0))],
        out_specs=pl.BlockSpec((_BR, do), lambda i: (i, 0)),
        out_shape=jax.ShapeDtypeStruct((n, do), jnp.float32),
    )(p, w)


def _tanh_body(p_ref, o_ref):
    o_ref[...] = jnp.tanh(p_ref[0] + p_ref[1])


def _addtanh(p, n):
    d = p.shape[2]
    return pl.pallas_call(
        _tanh_body,
        grid=(n // _BR,),
        in_specs=[pl.BlockSpec((2, _BR, d), lambda i: (0, i, 0))],
        out_specs=pl.BlockSpec((_BR, d), lambda i: (i, 0)),
        out_shape=jax.ShapeDtypeStruct((n, d), jnp.float32),
    )(p)


def _mm_tanh_body(p_ref, w_ref, o_ref):
    h = lax.dot_general(
        p_ref[0] + p_ref[1], w_ref[...], (((1,), (1,)), ((), ())),
        preferred_element_type=jnp.float32)
    o_ref[...] = jnp.tanh(h)


def _add_matmul_tanh(p, w, n):
    d = p.shape[2]
    do = w.shape[0]
    return pl.pallas_call(
        _mm_tanh_body,
        grid=(n // _BR,),
        in_specs=[pl.BlockSpec((2, _BR, d), lambda i: (0, i, 0)),
                  pl.BlockSpec((do, d), lambda i: (0, 0))],
        out_specs=pl.BlockSpec((_BR, do), lambda i: (i, 0)),
        out_shape=jax.ShapeDtypeStruct((n, do), jnp.float32),
    )(p, w)


def kernel(x, edge_index, edge_weight, W0, W1, W2):
    src = edge_index[1].astype(jnp.int32)
    dst = edge_index[0].astype(jnp.int32)
    w = edge_weight.astype(jnp.float32)
    pad = E_PAD - N_EDGES
    src3 = jnp.concatenate([src, jnp.zeros((pad,), jnp.int32)]).reshape(NW, N_CHUNKS, CHUNK)
    dst3 = jnp.concatenate([dst, jnp.zeros((pad,), jnp.int32)]).reshape(NW, N_CHUNKS, CHUNK)
    w3 = jnp.concatenate([w, jnp.zeros((pad,), jnp.float32)]).reshape(NW, N_CHUNKS, CHUNK)

    t = _matmul(x, W0)
    p = _spmm(t, src3, dst3, w3)
    t = _addtanh_matmul(p, W1)
    p = _spmm(t, src3, dst3, w3)
    t = _addtanh(p, N_NODES)
    p = _spmm(t, src3, dst3, w3)
    return _add_matmul_tanh(p, W2, N_NODES)
